# Initial kernel scaffold; baseline (speedup 1.0000x reference)
import jax
import jax.numpy as jnp
from jax.experimental import pallas as pl


def kernel(input_feature, attention_mask, prev_m, W_L, W1, W2, W3):
    raise NotImplementedError("write your pallas kernel here")


# speedup vs baseline: 5.2880x; 5.2880x over previous
import functools

import jax
import jax.numpy as jnp
from jax.experimental import pallas as pl

_TOK_BLOCK = 1024

_INV_SQRT2 = 0.7071067811865476


def _gelu_exact(x):
    return 0.5 * x * (1.0 + jax.lax.erf(x * _INV_SQRT2))


def _fused_kernel(x_ref, pm_ref, wl_ref, w1_ref, w2_ref, w3_ref,
                  out_ref, mask_ref, curr_ref):
    x0 = x_ref[...]

    def dot(a, b):
        return jax.lax.dot_general(
            a.astype(jnp.bfloat16), b.astype(jnp.bfloat16),
            dimension_numbers=(((1,), (1,)), ((), ())),
            preferred_element_type=jnp.float32,
        )
    h = _gelu_exact(dot(x0, wl_ref[...]))
    h = _gelu_exact(dot(h, w1_ref[...]))
    h = _gelu_exact(dot(h, w2_ref[...]))
    logit = dot(h, w3_ref[...])[:, 0]

    curr = jax.nn.sigmoid(logit) * pm_ref[0, 0, :]
    keep = (curr > 0.5).astype(jnp.float32)
    curr_m = keep + 1e-10
    curr_ref[0, 0, :] = curr_m
    mask_ref[0, 0, :] = curr_m.astype(jnp.int32)
    out_ref[...] = x0 * curr_m[:, None]


def kernel(input_feature, attention_mask, prev_m, W_L, W1, W2, W3):
    B, S, D = input_feature.shape
    N = B * S
    T = _TOK_BLOCK
    grid = (N // T,)

    x = input_feature.reshape(N, D)
    pm = prev_m.reshape(N // T, 1, T)
    W3p = jnp.concatenate([W3, jnp.zeros((7, W3.shape[1]), W3.dtype)], axis=0)

    out, mask, curr_m = pl.pallas_call(
        _fused_kernel,
        grid=grid,
        in_specs=[
            pl.BlockSpec((T, D), lambda i: (i, 0)),
            pl.BlockSpec((1, 1, T), lambda i: (i, 0, 0)),
            pl.BlockSpec(W_L.shape, lambda i: (0, 0)),
            pl.BlockSpec(W1.shape, lambda i: (0, 0)),
            pl.BlockSpec(W2.shape, lambda i: (0, 0)),
            pl.BlockSpec(W3p.shape, lambda i: (0, 0)),
        ],
        out_specs=[
            pl.BlockSpec((T, D), lambda i: (i, 0)),
            pl.BlockSpec((1, 1, T), lambda i: (i, 0, 0)),
            pl.BlockSpec((1, 1, T), lambda i: (i, 0, 0)),
        ],
        out_shape=[
            jax.ShapeDtypeStruct((N, D), jnp.float32),
            jax.ShapeDtypeStruct((N // T, 1, T), jnp.int32),
            jax.ShapeDtypeStruct((N // T, 1, T), jnp.float32),
        ],
    )(x, pm, W_L, W1, W2, W3p)

    return (out.reshape(B, S, D), mask.reshape(B, S), curr_m.reshape(B, S))
